# baseline (speedup 1.0000x reference)
import functools

import jax
import jax.numpy as jnp
from jax import lax
from jax.experimental import pallas as pl
from jax.experimental.pallas import tpu as pltpu
from jax.experimental.pallas import tpu_sc as plsc

N = 10000
E = 320000
D = 128
C = 2

NC = 2
NS = 16
NW = NC * NS
EC = 128
NCHUNK = 79
EPT = NCHUNK * EC
EPAD = NW * EPT - E
DGC = 64
DGT = 160
DGPAD = NW * DGT * DGC - E
NP = 10240
RPT = NP // NS


_mesh = plsc.VectorSubcoreMesh(core_axis_name="c", subcore_axis_name="s")


@functools.partial(
    pl.kernel,
    mesh=_mesh,
    out_type=jax.ShapeDtypeStruct((NC, NP), jnp.float32),
    scratch_types=[
        pltpu.VMEM((DGT, DGC), jnp.int32),
        pltpu.VMEM((DGC,), jnp.float32),
        pltpu.VMEM((RPT,), jnp.float32),
        pltpu.VMEM_SHARED((NP,), jnp.float32),
    ],
)
def _deg(dstp_hbm, out_hbm, dst_v, ones_v, zbuf_v, acc_sh):
    cid = lax.axis_index("c")
    sid = lax.axis_index("s")
    wid = cid * NS + sid

    one16 = jnp.ones((16,), jnp.float32)
    for j in range(DGC // 16):
        ones_v[pl.ds(j * 16, 16)] = one16
    z16 = jnp.zeros((16,), jnp.float32)

    def zb(i, carry):
        zbuf_v[pl.ds(i * 16, 16)] = z16
        return carry

    lax.fori_loop(0, RPT // 16, zb, 0)
    pltpu.sync_copy(zbuf_v, acc_sh.at[pl.ds(sid * RPT, RPT)])
    pltpu.sync_copy(dstp_hbm.at[wid], dst_v)
    plsc.subcore_barrier()

    def chunk(j, carry):
        pltpu.sync_copy(ones_v, acc_sh.at[dst_v.at[j]], add=True)
        return carry

    lax.fori_loop(0, DGT, chunk, 0)
    plsc.subcore_barrier()
    pltpu.sync_copy(acc_sh.at[pl.ds(sid * RPT, RPT)],
                    out_hbm.at[cid, pl.ds(sid * RPT, RPT)])


@functools.partial(
    pl.kernel,
    mesh=_mesh,
    out_type=jax.ShapeDtypeStruct((NC, NP, D), jnp.float32),
    scratch_types=[
        pltpu.VMEM((EC,), jnp.int32),
        pltpu.VMEM((EC,), jnp.int32),
        pltpu.VMEM((EC,), jnp.int32),
        pltpu.VMEM((EC,), jnp.int32),
        pltpu.VMEM((EC, D), jnp.float32),
        pltpu.VMEM((EC, D), jnp.float32),
        pltpu.VMEM_SHARED((NP, D), jnp.float32),
        pltpu.SemaphoreType.DMA,
        pltpu.SemaphoreType.DMA,
    ],
)
def _agg(hs_hbm, src_hbm, dst_hbm, out_hbm, src_a, src_b, dst_a, dst_b,
         rows_a, rows_b, acc_sh, sem_a, sem_b):
    cid = lax.axis_index("c")
    sid = lax.axis_index("s")
    wid = cid * NS + sid

    z16 = jnp.zeros((16,), jnp.float32)

    def zrow(i, carry):
        for j in range(D // 16):
            rows_a[i, pl.ds(j * 16, 16)] = z16
        return carry

    lax.fori_loop(0, EC, zrow, 0)

    row0 = sid * RPT
    for r in range(RPT // EC):
        pltpu.sync_copy(rows_a, acc_sh.at[pl.ds(row0 + r * EC, EC)])
    plsc.subcore_barrier()

    ebase = wid * EPT

    def load_idx(c, src_v, dst_v):
        off = ebase + c * EC
        pltpu.sync_copy(src_hbm.at[pl.ds(off, EC)], src_v)
        pltpu.sync_copy(dst_hbm.at[pl.ds(off, EC)], dst_v)

    def gather_start(src_v, rows, sem):
        pltpu.async_copy(hs_hbm.at[src_v], rows, sem)

    def gather_wait(src_v, rows, sem):
        pltpu.make_async_copy(hs_hbm.at[src_v], rows, sem).wait()

    def scatter(dst_v, rows):
        pltpu.sync_copy(rows, acc_sh.at[dst_v], add=True)

    load_idx(0, src_a, dst_a)
    gather_start(src_a, rows_a, sem_a)

    def body(j, carry):
        c = 2 * j
        load_idx(c + 1, src_b, dst_b)
        gather_start(src_b, rows_b, sem_b)
        gather_wait(src_a, rows_a, sem_a)
        scatter(dst_a, rows_a)
        load_idx(c + 2, src_a, dst_a)
        gather_start(src_a, rows_a, sem_a)
        gather_wait(src_b, rows_b, sem_b)
        scatter(dst_b, rows_b)
        return carry

    lax.fori_loop(0, NCHUNK // 2, body, 0)
    gather_wait(src_a, rows_a, sem_a)
    scatter(dst_a, rows_a)

    plsc.subcore_barrier()
    pltpu.sync_copy(acc_sh.at[pl.ds(row0, RPT)],
                    out_hbm.at[cid, pl.ds(row0, RPT)])


BLK = 1000
GRID = N // BLK
_HI = lax.Precision.HIGHEST


def _tc1_body(degT_ref, x_ref, w1_ref, dis_ref, hs1_ref):
    deg = degT_ref[...]
    dis = lax.rsqrt(deg[:, 0:1] + deg[:, 1:2] + 1.0)
    dis_ref[...] = dis
    mm = jnp.dot(x_ref[...], w1_ref[...], preferred_element_type=jnp.float32,
                 precision=_HI)
    hs1_ref[...] = mm * dis


_tc1 = pl.pallas_call(
    _tc1_body,
    grid=(GRID,),
    in_specs=[
        pl.BlockSpec((BLK, NC), lambda i: (i, 0)),
        pl.BlockSpec((BLK, D), lambda i: (i, 0)),
        pl.BlockSpec((D, D), lambda i: (0, 0)),
    ],
    out_specs=[
        pl.BlockSpec((BLK, 1), lambda i: (i, 0)),
        pl.BlockSpec((BLK, D), lambda i: (i, 0)),
    ],
    out_shape=[
        jax.ShapeDtypeStruct((N, 1), jnp.float32),
        jax.ShapeDtypeStruct((N, D), jnp.float32),
    ],
)


def _tc2_body(a0_ref, a1_ref, hs1_ref, dis_ref, b1_ref, w2_ref, hs2_ref):
    dis = dis_ref[...]
    t = a0_ref[...] + a1_ref[...] + hs1_ref[...]
    h1 = jnp.maximum(dis * t + b1_ref[...], 0.0)
    hs2_ref[...] = jnp.dot(h1, w2_ref[...], preferred_element_type=jnp.float32,
                           precision=_HI) * dis


_tc2 = pl.pallas_call(
    _tc2_body,
    grid=(GRID,),
    in_specs=[
        pl.BlockSpec((BLK, D), lambda i: (i, 0)),
        pl.BlockSpec((BLK, D), lambda i: (i, 0)),
        pl.BlockSpec((BLK, D), lambda i: (i, 0)),
        pl.BlockSpec((BLK, 1), lambda i: (i, 0)),
        pl.BlockSpec((1, D), lambda i: (0, 0)),
        pl.BlockSpec((D, D), lambda i: (0, 0)),
    ],
    out_specs=pl.BlockSpec((BLK, D), lambda i: (i, 0)),
    out_shape=jax.ShapeDtypeStruct((N, D), jnp.float32),
)


def _tc3_body(a0_ref, a1_ref, hs2_ref, dis_ref, b2_ref, wc_ref, bc_ref,
              logits_ref, h2_ref):
    dis = dis_ref[...]
    t = a0_ref[...] + a1_ref[...] + hs2_ref[...]
    h2 = jnp.maximum(dis * t + b2_ref[...], 0.0)
    h2_ref[...] = h2
    logits_ref[...] = jnp.dot(h2, wc_ref[...],
                              preferred_element_type=jnp.float32,
                              precision=_HI) + bc_ref[...]


_tc3 = pl.pallas_call(
    _tc3_body,
    grid=(GRID,),
    in_specs=[
        pl.BlockSpec((BLK, D), lambda i: (i, 0)),
        pl.BlockSpec((BLK, D), lambda i: (i, 0)),
        pl.BlockSpec((BLK, D), lambda i: (i, 0)),
        pl.BlockSpec((BLK, 1), lambda i: (i, 0)),
        pl.BlockSpec((1, D), lambda i: (0, 0)),
        pl.BlockSpec((D, C), lambda i: (0, 0)),
        pl.BlockSpec((1, C), lambda i: (0, 0)),
    ],
    out_specs=[
        pl.BlockSpec((BLK, C), lambda i: (i, 0)),
        pl.BlockSpec((BLK, D), lambda i: (i, 0)),
    ],
    out_shape=[
        jax.ShapeDtypeStruct((N, C), jnp.float32),
        jax.ShapeDtypeStruct((N, D), jnp.float32),
    ],
)


def kernel(x, edge_index, W1, b1, W2, b2, Wc, bc):
    ei = edge_index.astype(jnp.int32)
    src = ei[0]
    dst = ei[1]
    pad_src = jnp.zeros((EPAD,), jnp.int32)
    pad_dst = N + (jnp.arange(EPAD, dtype=jnp.int32) % (NP - N))
    srcp = jnp.concatenate([src, pad_src])
    dstp = jnp.concatenate([dst, pad_dst])
    dpad_dst = N + (jnp.arange(DGPAD, dtype=jnp.int32) % (NP - N))
    dplane = jnp.concatenate([dst, dpad_dst]).reshape(NW, DGT, DGC)
    degp = _deg(dplane)
    dis, hs1 = _tc1(degp[:, :N].T, x, W1)
    agg1 = _agg(hs1, srcp, dstp)
    hs2 = _tc2(agg1[0, :N], agg1[1, :N], hs1, dis, b1.reshape(1, D), W2)
    agg2 = _agg(hs2, srcp, dstp)
    logits, h2 = _tc3(agg2[0, :N], agg2[1, :N], hs2, dis, b2.reshape(1, D),
                      Wc, bc.reshape(1, C))
    return (logits, h2)


# speedup vs baseline: 1.5441x; 1.5441x over previous
import functools

import jax
import jax.numpy as jnp
from jax import lax
from jax.experimental import pallas as pl
from jax.experimental.pallas import tpu as pltpu
from jax.experimental.pallas import tpu_sc as plsc

N = 10000
E = 320000
D = 128
C = 2

NC = 2
NS = 16
NW = NC * NS
EC = 80
EPT = E // NW
NCHUNK = EPT // EC
DGC = 64
DGT = 160
DGPAD = NW * DGT * DGC - E
NP = 10240
RPT = NP // NS


_mesh = plsc.VectorSubcoreMesh(core_axis_name="c", subcore_axis_name="s")


@functools.partial(
    pl.kernel,
    mesh=_mesh,
    out_type=jax.ShapeDtypeStruct((NC, NP), jnp.float32),
    scratch_types=[
        pltpu.VMEM((DGT, DGC), jnp.int32),
        pltpu.VMEM((DGC,), jnp.float32),
        pltpu.VMEM((RPT,), jnp.float32),
        pltpu.VMEM_SHARED((NP,), jnp.float32),
    ],
)
def _deg(dstp_hbm, out_hbm, dst_v, ones_v, zbuf_v, acc_sh):
    cid = lax.axis_index("c")
    sid = lax.axis_index("s")
    wid = cid * NS + sid

    one16 = jnp.ones((16,), jnp.float32)
    for j in range(DGC // 16):
        ones_v[pl.ds(j * 16, 16)] = one16
    z16 = jnp.zeros((16,), jnp.float32)

    def zb(i, carry):
        zbuf_v[pl.ds(i * 16, 16)] = z16
        return carry

    lax.fori_loop(0, RPT // 16, zb, 0)
    pltpu.sync_copy(zbuf_v, acc_sh.at[pl.ds(sid * RPT, RPT)])
    pltpu.sync_copy(dstp_hbm.at[wid], dst_v)
    plsc.subcore_barrier()

    def chunk(j, carry):
        pltpu.sync_copy(ones_v, acc_sh.at[dst_v.at[j]], add=True)
        return carry

    lax.fori_loop(0, DGT, chunk, 0)
    plsc.subcore_barrier()
    pltpu.sync_copy(acc_sh.at[pl.ds(sid * RPT, RPT)],
                    out_hbm.at[cid, pl.ds(sid * RPT, RPT)])


@functools.partial(
    pl.kernel,
    mesh=_mesh,
    out_type=jax.ShapeDtypeStruct((NC, NP, D), jnp.float32),
    scratch_types=[
        pltpu.VMEM((EC,), jnp.int32),
        pltpu.VMEM((EC,), jnp.int32),
        pltpu.VMEM((EC,), jnp.int32),
        pltpu.VMEM((EC,), jnp.int32),
        pltpu.VMEM((EC, D), jnp.float32),
        pltpu.VMEM((EC, D), jnp.float32),
        pltpu.VMEM_SHARED((NP, D), jnp.float32),
        pltpu.SemaphoreType.DMA,
        pltpu.SemaphoreType.DMA,
    ],
)
def _agg(hs_hbm, src_hbm, dst_hbm, out_hbm, src_a, src_b, dst_a, dst_b,
         rows_a, rows_b, acc_sh, sem_a, sem_b):
    cid = lax.axis_index("c")
    sid = lax.axis_index("s")
    wid = cid * NS + sid

    z16 = jnp.zeros((16,), jnp.float32)

    def zrow(i, carry):
        for j in range(D // 16):
            rows_a[i, pl.ds(j * 16, 16)] = z16
        return carry

    lax.fori_loop(0, EC, zrow, 0)

    row0 = sid * RPT
    for r in range(RPT // EC):
        pltpu.sync_copy(rows_a, acc_sh.at[pl.ds(row0 + r * EC, EC)])
    plsc.subcore_barrier()

    ebase = wid * EPT

    def load_idx(c, src_v, dst_v):
        off = ebase + c * EC
        pltpu.sync_copy(src_hbm.at[pl.ds(off, EC)], src_v)
        pltpu.sync_copy(dst_hbm.at[pl.ds(off, EC)], dst_v)

    def gather_start(src_v, rows, sem):
        pltpu.async_copy(hs_hbm.at[src_v], rows, sem)

    def gather_wait(src_v, rows, sem):
        pltpu.make_async_copy(hs_hbm.at[src_v], rows, sem).wait()

    def scatter(dst_v, rows):
        pltpu.sync_copy(rows, acc_sh.at[dst_v], add=True)

    load_idx(0, src_a, dst_a)
    gather_start(src_a, rows_a, sem_a)

    def body(j, carry):
        c = 2 * j
        load_idx(c + 1, src_b, dst_b)
        gather_start(src_b, rows_b, sem_b)
        gather_wait(src_a, rows_a, sem_a)
        scatter(dst_a, rows_a)
        load_idx(c + 2, src_a, dst_a)
        gather_start(src_a, rows_a, sem_a)
        gather_wait(src_b, rows_b, sem_b)
        scatter(dst_b, rows_b)
        return carry

    lax.fori_loop(0, NCHUNK // 2, body, 0)
    gather_wait(src_a, rows_a, sem_a)
    scatter(dst_a, rows_a)

    plsc.subcore_barrier()
    pltpu.sync_copy(acc_sh.at[pl.ds(row0, RPT)],
                    out_hbm.at[cid, pl.ds(row0, RPT)])


BLK = 1000
GRID = N // BLK
_HI = lax.Precision.HIGHEST


def _tc1_body(degT_ref, x_ref, w1_ref, dis_ref, hs1_ref):
    deg = degT_ref[...]
    dis = lax.rsqrt(deg[:, 0:1] + deg[:, 1:2] + 1.0)
    dis_ref[...] = dis
    mm = jnp.dot(x_ref[...], w1_ref[...], preferred_element_type=jnp.float32,
                 precision=_HI)
    hs1_ref[...] = mm * dis


_tc1 = pl.pallas_call(
    _tc1_body,
    grid=(GRID,),
    in_specs=[
        pl.BlockSpec((BLK, NC), lambda i: (i, 0)),
        pl.BlockSpec((BLK, D), lambda i: (i, 0)),
        pl.BlockSpec((D, D), lambda i: (0, 0)),
    ],
    out_specs=[
        pl.BlockSpec((BLK, 1), lambda i: (i, 0)),
        pl.BlockSpec((BLK, D), lambda i: (i, 0)),
    ],
    out_shape=[
        jax.ShapeDtypeStruct((N, 1), jnp.float32),
        jax.ShapeDtypeStruct((N, D), jnp.float32),
    ],
)


def _tc2_body(a0_ref, a1_ref, hs1_ref, dis_ref, b1_ref, w2_ref, hs2_ref):
    dis = dis_ref[...]
    t = a0_ref[...] + a1_ref[...] + hs1_ref[...]
    h1 = jnp.maximum(dis * t + b1_ref[...], 0.0)
    hs2_ref[...] = jnp.dot(h1, w2_ref[...], preferred_element_type=jnp.float32,
                           precision=_HI) * dis


_tc2 = pl.pallas_call(
    _tc2_body,
    grid=(GRID,),
    in_specs=[
        pl.BlockSpec((BLK, D), lambda i: (i, 0)),
        pl.BlockSpec((BLK, D), lambda i: (i, 0)),
        pl.BlockSpec((BLK, D), lambda i: (i, 0)),
        pl.BlockSpec((BLK, 1), lambda i: (i, 0)),
        pl.BlockSpec((1, D), lambda i: (0, 0)),
        pl.BlockSpec((D, D), lambda i: (0, 0)),
    ],
    out_specs=pl.BlockSpec((BLK, D), lambda i: (i, 0)),
    out_shape=jax.ShapeDtypeStruct((N, D), jnp.float32),
)


def _tc3_body(a0_ref, a1_ref, hs2_ref, dis_ref, b2_ref, wc_ref, bc_ref,
              logits_ref, h2_ref):
    dis = dis_ref[...]
    t = a0_ref[...] + a1_ref[...] + hs2_ref[...]
    h2 = jnp.maximum(dis * t + b2_ref[...], 0.0)
    h2_ref[...] = h2
    logits_ref[...] = jnp.dot(h2, wc_ref[...],
                              preferred_element_type=jnp.float32,
                              precision=_HI) + bc_ref[...]


_tc3 = pl.pallas_call(
    _tc3_body,
    grid=(GRID,),
    in_specs=[
        pl.BlockSpec((BLK, D), lambda i: (i, 0)),
        pl.BlockSpec((BLK, D), lambda i: (i, 0)),
        pl.BlockSpec((BLK, D), lambda i: (i, 0)),
        pl.BlockSpec((BLK, 1), lambda i: (i, 0)),
        pl.BlockSpec((1, D), lambda i: (0, 0)),
        pl.BlockSpec((D, C), lambda i: (0, 0)),
        pl.BlockSpec((1, C), lambda i: (0, 0)),
    ],
    out_specs=[
        pl.BlockSpec((BLK, C), lambda i: (i, 0)),
        pl.BlockSpec((BLK, D), lambda i: (i, 0)),
    ],
    out_shape=[
        jax.ShapeDtypeStruct((N, C), jnp.float32),
        jax.ShapeDtypeStruct((N, D), jnp.float32),
    ],
)


def kernel(x, edge_index, W1, b1, W2, b2, Wc, bc):
    ei = edge_index.astype(jnp.int32)
    src = ei[0]
    dst = ei[1]
    pad_dst = N + (jnp.arange(DGPAD, dtype=jnp.int32) % (NP - N))
    dstp = jnp.concatenate([dst, pad_dst]).reshape(NW, DGT, DGC)
    degp = _deg(dstp)
    dis, hs1 = _tc1(degp[:, :N].T, x, W1)
    agg1 = _agg(hs1, src, dst)
    hs2 = _tc2(agg1[0, :N], agg1[1, :N], hs1, dis, b1.reshape(1, D), W2)
    agg2 = _agg(hs2, src, dst)
    logits, h2 = _tc3(agg2[0, :N], agg2[1, :N], hs2, dis, b2.reshape(1, D),
                      Wc, bc.reshape(1, C))
    return (logits, h2)


# speedup vs baseline: 1.6304x; 1.0559x over previous
import functools

import jax
import jax.numpy as jnp
from jax import lax
from jax.experimental import pallas as pl
from jax.experimental.pallas import tpu as pltpu
from jax.experimental.pallas import tpu_sc as plsc

N = 10000
E = 320000
D = 128
C = 2

NC = 2
NS = 16
NW = NC * NS
EC = 80
EPT = E // NW
NCHUNK = EPT // EC
DGC = 64
DGT = 160
DGPAD = NW * DGT * DGC - E
NP = 10240
RPT = NP // NS


_mesh = plsc.VectorSubcoreMesh(core_axis_name="c", subcore_axis_name="s")


@functools.partial(
    pl.kernel,
    mesh=_mesh,
    out_type=jax.ShapeDtypeStruct((NC, NP), jnp.float32),
    scratch_types=[
        pltpu.VMEM((DGT, DGC), jnp.int32),
        pltpu.VMEM((DGC,), jnp.float32),
        pltpu.VMEM((RPT,), jnp.float32),
        pltpu.VMEM_SHARED((NP,), jnp.float32),
    ],
)
def _deg(dstp_hbm, out_hbm, dst_v, ones_v, zbuf_v, acc_sh):
    cid = lax.axis_index("c")
    sid = lax.axis_index("s")
    wid = cid * NS + sid

    one16 = jnp.ones((16,), jnp.float32)
    for j in range(DGC // 16):
        ones_v[pl.ds(j * 16, 16)] = one16
    z16 = jnp.zeros((16,), jnp.float32)

    def zb(i, carry):
        zbuf_v[pl.ds(i * 16, 16)] = z16
        return carry

    lax.fori_loop(0, RPT // 16, zb, 0)
    pltpu.sync_copy(zbuf_v, acc_sh.at[pl.ds(sid * RPT, RPT)])
    pltpu.sync_copy(dstp_hbm.at[wid], dst_v)
    plsc.subcore_barrier()

    def chunk(j, carry):
        pltpu.sync_copy(ones_v, acc_sh.at[dst_v.at[j]], add=True)
        return carry

    lax.fori_loop(0, DGT, chunk, 0)
    plsc.subcore_barrier()
    pltpu.sync_copy(acc_sh.at[pl.ds(sid * RPT, RPT)],
                    out_hbm.at[cid, pl.ds(sid * RPT, RPT)])


@functools.partial(
    pl.kernel,
    mesh=_mesh,
    out_type=jax.ShapeDtypeStruct((NC, NP, D), jnp.float32),
    scratch_types=[
        pltpu.VMEM((EC,), jnp.int32),
        pltpu.VMEM((EC,), jnp.int32),
        pltpu.VMEM((EC,), jnp.int32),
        pltpu.VMEM((EC,), jnp.int32),
        pltpu.VMEM((EC, D), jnp.float32),
        pltpu.VMEM((EC, D), jnp.float32),
        pltpu.VMEM_SHARED((NP, D), jnp.float32),
        pltpu.SemaphoreType.DMA,
        pltpu.SemaphoreType.DMA,
    ],
)
def _agg(hs_hbm, src_hbm, dst_hbm, out_hbm, src_a, src_b, dst_a, dst_b,
         rows_a, rows_b, acc_sh, sem_a, sem_b):
    cid = lax.axis_index("c")
    sid = lax.axis_index("s")
    wid = cid * NS + sid

    z16 = jnp.zeros((16,), jnp.float32)

    def zrow(i, carry):
        for j in range(D // 16):
            rows_a[i, pl.ds(j * 16, 16)] = z16
        return carry

    lax.fori_loop(0, EC, zrow, 0)

    row0 = sid * RPT
    for r in range(RPT // EC):
        pltpu.sync_copy(rows_a, acc_sh.at[pl.ds(row0 + r * EC, EC)])
    plsc.subcore_barrier()

    ebase = wid * EPT

    def load_idx(c, src_v, dst_v):
        off = ebase + c * EC
        pltpu.sync_copy(src_hbm.at[pl.ds(off, EC)], src_v)
        pltpu.sync_copy(dst_hbm.at[pl.ds(off, EC)], dst_v)

    def gather_start(src_v, rows, sem):
        pltpu.async_copy(hs_hbm.at[src_v], rows, sem)

    def gather_wait(src_v, rows, sem):
        pltpu.make_async_copy(hs_hbm.at[src_v], rows, sem).wait()

    def scatter(dst_v, rows):
        pltpu.sync_copy(rows, acc_sh.at[dst_v], add=True)

    load_idx(0, src_a, dst_a)
    gather_start(src_a, rows_a, sem_a)

    def body(j, carry):
        c = 2 * j
        load_idx(c + 1, src_b, dst_b)
        gather_start(src_b, rows_b, sem_b)
        gather_wait(src_a, rows_a, sem_a)
        scatter(dst_a, rows_a)
        load_idx(c + 2, src_a, dst_a)
        gather_start(src_a, rows_a, sem_a)
        gather_wait(src_b, rows_b, sem_b)
        scatter(dst_b, rows_b)
        return carry

    lax.fori_loop(0, NCHUNK // 2, body, 0)
    gather_wait(src_a, rows_a, sem_a)
    scatter(dst_a, rows_a)

    plsc.subcore_barrier()
    pltpu.sync_copy(acc_sh.at[pl.ds(row0, RPT)],
                    out_hbm.at[cid, pl.ds(row0, RPT)])


BLK = 2000
GRID = N // BLK
_HI = lax.Precision.HIGHEST


def _tc1_body(degT_ref, x_ref, w1_ref, dis_ref, hs1_ref):
    deg = degT_ref[...]
    dis = lax.rsqrt(deg[:, 0:1] + deg[:, 1:2] + 1.0)
    dis_ref[...] = dis
    mm = jnp.dot(x_ref[...], w1_ref[...], preferred_element_type=jnp.float32,
                 precision=_HI)
    hs1_ref[...] = mm * dis


_tc1 = pl.pallas_call(
    _tc1_body,
    grid=(GRID,),
    in_specs=[
        pl.BlockSpec((BLK, NC), lambda i: (i, 0)),
        pl.BlockSpec((BLK, D), lambda i: (i, 0)),
        pl.BlockSpec((D, D), lambda i: (0, 0)),
    ],
    out_specs=[
        pl.BlockSpec((BLK, 1), lambda i: (i, 0)),
        pl.BlockSpec((BLK, D), lambda i: (i, 0)),
    ],
    out_shape=[
        jax.ShapeDtypeStruct((N, 1), jnp.float32),
        jax.ShapeDtypeStruct((N, D), jnp.float32),
    ],
)


def _tc2_body(a0_ref, a1_ref, hs1_ref, dis_ref, b1_ref, w2_ref, hs2_ref):
    dis = dis_ref[...]
    t = a0_ref[0] + a1_ref[0] + hs1_ref[...]
    h1 = jnp.maximum(dis * t + b1_ref[...], 0.0)
    hs2_ref[...] = jnp.dot(h1, w2_ref[...], preferred_element_type=jnp.float32,
                           precision=_HI) * dis


_tc2 = pl.pallas_call(
    _tc2_body,
    grid=(GRID,),
    in_specs=[
        pl.BlockSpec((1, BLK, D), lambda i: (0, i, 0)),
        pl.BlockSpec((1, BLK, D), lambda i: (1, i, 0)),
        pl.BlockSpec((BLK, D), lambda i: (i, 0)),
        pl.BlockSpec((BLK, 1), lambda i: (i, 0)),
        pl.BlockSpec((1, D), lambda i: (0, 0)),
        pl.BlockSpec((D, D), lambda i: (0, 0)),
    ],
    out_specs=pl.BlockSpec((BLK, D), lambda i: (i, 0)),
    out_shape=jax.ShapeDtypeStruct((N, D), jnp.float32),
)


def _tc3_body(a0_ref, a1_ref, hs2_ref, dis_ref, b2_ref, wc_ref, bc_ref,
              logits_ref, h2_ref):
    dis = dis_ref[...]
    t = a0_ref[0] + a1_ref[0] + hs2_ref[...]
    h2 = jnp.maximum(dis * t + b2_ref[...], 0.0)
    h2_ref[...] = h2
    logits_ref[...] = jnp.dot(h2, wc_ref[...],
                              preferred_element_type=jnp.float32,
                              precision=_HI) + bc_ref[...]


_tc3 = pl.pallas_call(
    _tc3_body,
    grid=(GRID,),
    in_specs=[
        pl.BlockSpec((1, BLK, D), lambda i: (0, i, 0)),
        pl.BlockSpec((1, BLK, D), lambda i: (1, i, 0)),
        pl.BlockSpec((BLK, D), lambda i: (i, 0)),
        pl.BlockSpec((BLK, 1), lambda i: (i, 0)),
        pl.BlockSpec((1, D), lambda i: (0, 0)),
        pl.BlockSpec((D, C), lambda i: (0, 0)),
        pl.BlockSpec((1, C), lambda i: (0, 0)),
    ],
    out_specs=[
        pl.BlockSpec((BLK, C), lambda i: (i, 0)),
        pl.BlockSpec((BLK, D), lambda i: (i, 0)),
    ],
    out_shape=[
        jax.ShapeDtypeStruct((N, C), jnp.float32),
        jax.ShapeDtypeStruct((N, D), jnp.float32),
    ],
)


def kernel(x, edge_index, W1, b1, W2, b2, Wc, bc):
    ei = edge_index.astype(jnp.int32)
    src = ei[0]
    dst = ei[1]
    pad_dst = N + (jnp.arange(DGPAD, dtype=jnp.int32) % (NP - N))
    dstp = jnp.concatenate([dst, pad_dst]).reshape(NW, DGT, DGC)
    degp = _deg(dstp)
    dis, hs1 = _tc1(degp[:, :N].T, x, W1)
    agg1 = _agg(hs1, src, dst)
    hs2 = _tc2(agg1, agg1, hs1, dis, b1.reshape(1, D), W2)
    agg2 = _agg(hs2, src, dst)
    logits, h2 = _tc3(agg2, agg2, hs2, dis, b2.reshape(1, D),
                      Wc, bc.reshape(1, C))
    return (logits, h2)
